# baseline (speedup 1.0000x reference)
import jax
import jax.numpy as jnp
from jax import lax
from jax.experimental import pallas as pl
from jax.experimental.pallas import tpu as pltpu
from jax.experimental.pallas import tpu_sc as plsc

_N = 32768
_N_ROWS = 128
_NC = 2
_NS = 16
_ROWS_PER_W = _N_ROWS // (_NC * _NS)
_INT_MIN = -(2**31)
_M = int(round(_N * 0.2)) - 1


def _ukey(v):
    b = plsc.bitcast(v, jnp.int32)
    m = (b >> 31) | jnp.int32(_INT_MIN)
    return plsc.bitcast(b ^ m, jnp.uint32)


def _unkey(u):
    bits = jnp.where(u >= jnp.uint32(2**31), u ^ jnp.uint32(2**31), ~u)
    return plsc.bitcast(bits, jnp.float32)


def _sc_kwta(x_hbm, o_hbm, row0, row1, cand, hist, folded, si0, si1, so):
    iota = lax.iota(jnp.int32, 16)
    ones = jnp.full((16,), 1, jnp.int32)
    zeros = jnp.zeros((16,), jnp.int32)
    wid = lax.axis_index("s") * _NC + lax.axis_index("c")

    @plsc.parallel_loop(0, 512, unroll=4)
    def _(i):
        hist[pl.ds(i * 16, 16)] = zeros

    def fold_and_scan(n_src, m_rem, nb):
        @plsc.parallel_loop(0, nb // 16)
        def _(c):
            acc = zeros
            for l in range(16):
                sl = pl.ds(l * 512 + c * 16, 16)
                acc = acc + hist[sl]
                hist[sl] = zeros
            folded[pl.ds(c * 16, 16)] = acc

        def scan(c, carry):
            pc, bst, pb, hb = carry
            h = folded[pl.ds(c * 16, 16)]
            cs = plsc.cumsum(h)
            pex = pc + cs - h
            cond = (n_src - pex) >= m_rem
            ll = jnp.max(jnp.where(cond, iota, -1))
            found = ll >= 0
            pbn = jnp.sum(jnp.where(iota == ll, pex, 0))
            hbn = jnp.sum(jnp.where(iota == ll, h, 0))
            bst = jnp.where(found, c * 16 + ll, bst)
            pb = jnp.where(found, pbn, pb)
            hb = jnp.where(found, hbn, hb)
            return pc + jnp.sum(h), bst, pb, hb

        zi = jnp.int32(0)
        _, bst, pb, hb = lax.fori_loop(0, nb // 16, scan, (zi, zi, zi, zi))
        return bst, m_rem - (n_src - pb - hb)

    def cand_round(n_src, m_rem, shift, maskval, nb, do_compact):
        nchunks = (n_src + 15) >> 4

        @plsc.parallel_loop(0, nchunks, unroll=4)
        def _(i):
            u = plsc.bitcast(cand[pl.ds(i * 16, 16)], jnp.uint32)
            bkt = jnp.right_shift(u, jnp.uint32(shift)).astype(jnp.int32) & maskval
            lanemask = iota < (n_src - i * 16)
            plsc.addupdate_scatter(hist, [iota * 512 + bkt], ones, mask=lanemask)

        bst, m_rem = fold_and_scan(n_src, m_rem, nb)
        if not do_compact:
            return bst, m_rem, n_src, None

        def compactc(i, carry):
            base, acc = carry
            v = cand[pl.ds(i * 16, 16)]
            u = plsc.bitcast(v, jnp.uint32)
            bkt = jnp.right_shift(u, jnp.uint32(shift)).astype(jnp.int32) & maskval
            lanemask = iota < (n_src - i * 16)
            msk = (bkt == bst) & lanemask
            mi = msk.astype(jnp.int32)
            pos = base + plsc.cumsum(mi) - mi
            plsc.store_scatter(cand, [pos], v, mask=msk)
            acc = acc + jnp.where((bkt > bst) & lanemask, _unkey(u), 0.0)
            return base + plsc.all_reduce_population_count(msk), acc

        zf16 = jnp.zeros((16,), jnp.float32)
        base, acc = plsc.parallel_loop(0, nchunks, unroll=4, carry=(zeros, zf16))(
            compactc
        )
        return bst, m_rem, jnp.max(base), acc

    def compute_thresh_inv(row):
        @plsc.parallel_loop(0, 2048, unroll=8)
        def _(i):
            u = _ukey(row[pl.ds(i * 16, 16)])
            bkt = jnp.right_shift(u, jnp.uint32(23)).astype(jnp.int32)
            plsc.addupdate_scatter(hist, [iota * 512 + bkt], ones)

        b1, m_rem = fold_and_scan(_N, _M, 512)

        zf16 = jnp.zeros((16,), jnp.float32)

        def compact1(i, carry):
            base, acc = carry
            v = row[pl.ds(i * 16, 16)]
            u = _ukey(v)
            bkt = jnp.right_shift(u, jnp.uint32(23)).astype(jnp.int32)
            msk = bkt == b1
            mi = msk.astype(jnp.int32)
            pos = base + plsc.cumsum(mi) - mi
            plsc.store_scatter(cand, [pos], plsc.bitcast(u, jnp.float32), mask=msk)
            acc = acc + jnp.where(bkt > b1, v, 0.0)
            return base + plsc.all_reduce_population_count(msk), acc

        base, acc1 = plsc.parallel_loop(0, 2048, unroll=8, carry=(zeros, zf16))(
            compact1
        )
        n1 = jnp.max(base)

        b2, m_rem, n2, acc2 = cand_round(n1, m_rem, 14, 511, 512, True)

        def small_path(_):
            v = plsc.bitcast(cand[pl.ds(0, 16)], jnp.uint32)
            sk, _, _ = plsc.sort_key_val(v, v, mask=iota < n2, descending=True)
            ki = jnp.sum(jnp.where(iota == m_rem - 1, plsc.bitcast(sk, jnp.int32), 0))
            acc = jnp.where(iota < m_rem - 1, _unkey(sk), 0.0)
            return ki, acc, jnp.int32(_M - 1)

        def big_path(_):
            b3, m_rem3, n3, acc3 = cand_round(n2, m_rem, 5, 511, 512, True)
            b4, m_rem4, _, _ = cand_round(n3, m_rem3, 0, 31, 32, False)
            ki = (b1 << 23) | (b2 << 14) | (b3 << 5) | b4
            kuv = plsc.bitcast(jnp.full((16,), ki), jnp.uint32)

            def tail(i, acc):
                u = plsc.bitcast(cand[pl.ds(i * 16, 16)], jnp.uint32)
                sel = (u > kuv) & (iota < (n3 - i * 16))
                return acc + jnp.where(sel, _unkey(u), 0.0)

            acc4 = plsc.parallel_loop(0, (n3 + 15) >> 4, unroll=4, carry=zf16)(tail)
            return ki, acc3 + acc4, _M - m_rem4

        ki, acc3, cnt = lax.cond(n2 <= 16, small_path, big_path, 0)
        kv = plsc.bitcast(jnp.full((16,), ki), jnp.uint32)
        tval = _unkey(kv)
        thresh = tval + jnp.float32(1e-8)

        s = jnp.sum(acc1 + acc2 + acc3)
        sv = jnp.full((16,), s) - cnt.astype(jnp.float32) * thresh
        inv = 1.0 / (sv + jnp.float32(1e-8))
        return thresh, inv

    def scale_to_cand(row, thresh, inv):
        @plsc.parallel_loop(0, 2048, unroll=8)
        def _(i):
            cand[pl.ds(i * 16, 16)] = (
                jnp.maximum(row[pl.ds(i * 16, 16)] - thresh, 0.0) * inv
            )

    rows = [row0, row1]
    sems = [si0, si1]
    r0 = wid * _ROWS_PER_W
    pltpu.make_async_copy(x_hbm.at[r0], row0, si0).start()
    pltpu.make_async_copy(x_hbm.at[r0 + 1], row1, si1).start()
    for j in range(_ROWS_PER_W):
        cur = rows[j % 2]
        pltpu.make_async_copy(x_hbm.at[r0 + j], cur, sems[j % 2]).wait()
        thresh, inv = compute_thresh_inv(cur)
        if j >= 1:
            pltpu.make_async_copy(cand, o_hbm.at[r0 + j - 1], so).wait()
        scale_to_cand(cur, thresh, inv)
        pltpu.make_async_copy(cand, o_hbm.at[r0 + j], so).start()
        if j + 2 < _ROWS_PER_W:
            pltpu.make_async_copy(x_hbm.at[r0 + j + 2], cur, sems[j % 2]).start()
    pltpu.make_async_copy(cand, o_hbm.at[r0 + _ROWS_PER_W - 1], so).wait()


@jax.jit
def kernel(x):
    mesh = plsc.VectorSubcoreMesh(
        core_axis_name="c", subcore_axis_name="s", num_cores=_NC, num_subcores=_NS
    )
    f = pl.kernel(
        _sc_kwta,
        out_type=jax.ShapeDtypeStruct((_N_ROWS, _N), jnp.float32),
        mesh=mesh,
        scratch_types=[
            pltpu.VMEM((_N,), jnp.float32),
            pltpu.VMEM((_N,), jnp.float32),
            pltpu.VMEM((_N,), jnp.float32),
            pltpu.VMEM((8192,), jnp.int32),
            pltpu.VMEM((512,), jnp.int32),
            pltpu.SemaphoreType.DMA,
            pltpu.SemaphoreType.DMA,
            pltpu.SemaphoreType.DMA,
        ],
        compiler_params=pltpu.CompilerParams(needs_layout_passes=False),
    )
    return f(x)


# speedup vs baseline: 1.0251x; 1.0251x over previous
import jax
import jax.numpy as jnp
from jax import lax
from jax.experimental import pallas as pl
from jax.experimental.pallas import tpu as pltpu
from jax.experimental.pallas import tpu_sc as plsc

_N = 32768
_N_ROWS = 128
_NC = 2
_NS = 16
_ROWS_PER_W = _N_ROWS // (_NC * _NS)
_INT_MIN = -(2**31)
_M = int(round(_N * 0.2)) - 1


def _ukey(v):
    b = plsc.bitcast(v, jnp.int32)
    m = (b >> 31) | jnp.int32(_INT_MIN)
    return plsc.bitcast(b ^ m, jnp.uint32)


def _unkey(u):
    bits = jnp.where(u >= jnp.uint32(2**31), u ^ jnp.uint32(2**31), ~u)
    return plsc.bitcast(bits, jnp.float32)


def _sc_kwta(x_hbm, o_hbm, row0, row1, cand, hist, folded, si0, si1, so):
    iota = lax.iota(jnp.int32, 16)
    ones = jnp.full((16,), 1, jnp.int32)
    zeros = jnp.zeros((16,), jnp.int32)
    wid = lax.axis_index("s") * _NC + lax.axis_index("c")

    @plsc.parallel_loop(0, 512, unroll=4)
    def _(i):
        hist[pl.ds(i * 16, 16)] = zeros

    lane15 = jnp.full((16,), 15, jnp.int32)

    def fold_and_scan(n_src, m_rem, nb):
        @plsc.parallel_loop(0, nb // 16)
        def _(c):
            acc = zeros
            for l in range(16):
                sl = pl.ds(l * 512 + c * 16, 16)
                acc = acc + hist[sl]
                hist[sl] = zeros
            folded[pl.ds(c * 16, 16)] = acc

        nv = jnp.full((16,), n_src, jnp.int32)

        def scan(c, carry):
            pc, bst, pb, hb = carry
            h = folded[pl.ds(c * 16, 16)]
            cs = plsc.cumsum(h)
            pex = pc + cs - h
            cond = (nv - pex) >= m_rem
            cpc = plsc.all_reduce_population_count(cond)
            found = cpc > 0
            ll = jnp.maximum(cpc - 1, 0)
            bst = jnp.where(found, c * 16 + ll, bst)
            pb = jnp.where(found, jnp.take(pex, ll), pb)
            hb = jnp.where(found, jnp.take(h, ll), hb)
            return pc + jnp.take(cs, lane15), bst, pb, hb

        zv = (zeros, zeros, zeros, zeros)
        _, bst, pb, hb = lax.fori_loop(0, nb // 16, scan, zv)
        return bst, m_rem - (nv - pb - hb)

    def cand_round(n_src, m_rem, shift, maskval, nb, do_compact):
        nchunks = (n_src + 15) >> 4

        @plsc.parallel_loop(0, nchunks, unroll=4)
        def _(i):
            u = plsc.bitcast(cand[pl.ds(i * 16, 16)], jnp.uint32)
            bkt = jnp.right_shift(u, jnp.uint32(shift)).astype(jnp.int32) & maskval
            lanemask = iota < (n_src - i * 16)
            plsc.addupdate_scatter(hist, [iota * 512 + bkt], ones, mask=lanemask)

        bst, m_rem = fold_and_scan(n_src, m_rem, nb)
        if not do_compact:
            return bst, m_rem, n_src, None

        def compactc(i, carry):
            base, acc = carry
            v = cand[pl.ds(i * 16, 16)]
            u = plsc.bitcast(v, jnp.uint32)
            bkt = jnp.right_shift(u, jnp.uint32(shift)).astype(jnp.int32) & maskval
            lanemask = iota < (n_src - i * 16)
            msk = (bkt == bst) & lanemask
            mi = msk.astype(jnp.int32)
            pos = base + plsc.cumsum(mi) - mi
            plsc.store_scatter(cand, [pos], v, mask=msk)
            acc = acc + jnp.where((bkt > bst) & lanemask, _unkey(u), 0.0)
            return base + plsc.all_reduce_population_count(msk), acc

        zf16 = jnp.zeros((16,), jnp.float32)
        base, acc = plsc.parallel_loop(0, nchunks, unroll=4, carry=(zeros, zf16))(
            compactc
        )
        return bst, m_rem, jnp.max(base), acc

    def compute_thresh_inv(row):
        @plsc.parallel_loop(0, 2048, unroll=8)
        def _(i):
            u = _ukey(row[pl.ds(i * 16, 16)])
            bkt = jnp.right_shift(u, jnp.uint32(23)).astype(jnp.int32)
            plsc.addupdate_scatter(hist, [iota * 512 + bkt], ones)

        b1, m_rem = fold_and_scan(_N, jnp.full((16,), _M, jnp.int32), 512)

        zf16 = jnp.zeros((16,), jnp.float32)

        def compact1(i, carry):
            base, acc = carry
            v = row[pl.ds(i * 16, 16)]
            u = _ukey(v)
            bkt = jnp.right_shift(u, jnp.uint32(23)).astype(jnp.int32)
            msk = bkt == b1
            mi = msk.astype(jnp.int32)
            pos = base + plsc.cumsum(mi) - mi
            plsc.store_scatter(cand, [pos], plsc.bitcast(u, jnp.float32), mask=msk)
            acc = acc + jnp.where(bkt > b1, v, 0.0)
            return base + plsc.all_reduce_population_count(msk), acc

        base, acc1 = plsc.parallel_loop(0, 2048, unroll=8, carry=(zeros, zf16))(
            compact1
        )
        n1 = jnp.max(base)

        b2, m_rem, n2, acc2 = cand_round(n1, m_rem, 14, 511, 512, True)

        def small_path(_):
            v = plsc.bitcast(cand[pl.ds(0, 16)], jnp.uint32)
            sk, _, _ = plsc.sort_key_val(v, v, mask=iota < n2, descending=True)
            ki = jnp.take(plsc.bitcast(sk, jnp.int32), jnp.maximum(m_rem - 1, 0))
            acc = jnp.where(iota < m_rem - 1, _unkey(sk), 0.0)
            return ki, acc, jnp.full((16,), _M - 1, jnp.int32)

        def big_path(_):
            b3, m_rem3, n3, acc3 = cand_round(n2, m_rem, 5, 511, 512, True)
            b4, m_rem4, _, _ = cand_round(n3, m_rem3, 0, 31, 32, False)
            ki = (b1 << 23) | (b2 << 14) | (b3 << 5) | b4
            kuv = plsc.bitcast(ki, jnp.uint32)

            def tail(i, acc):
                u = plsc.bitcast(cand[pl.ds(i * 16, 16)], jnp.uint32)
                sel = (u > kuv) & (iota < (n3 - i * 16))
                return acc + jnp.where(sel, _unkey(u), 0.0)

            acc4 = plsc.parallel_loop(0, (n3 + 15) >> 4, unroll=4, carry=zf16)(tail)
            return ki, acc3 + acc4, _M - m_rem4

        ki, acc3, cnt = lax.cond(n2 <= 16, small_path, big_path, 0)
        kv = plsc.bitcast(ki, jnp.uint32)
        tval = _unkey(kv)
        thresh = tval + jnp.float32(1e-8)

        s = jnp.sum(acc1 + acc2 + acc3)
        sv = jnp.full((16,), s) - cnt.astype(jnp.float32) * thresh
        inv = 1.0 / (sv + jnp.float32(1e-8))
        return thresh, inv

    def scale_to_cand(row, thresh, inv):
        @plsc.parallel_loop(0, 2048, unroll=8)
        def _(i):
            cand[pl.ds(i * 16, 16)] = (
                jnp.maximum(row[pl.ds(i * 16, 16)] - thresh, 0.0) * inv
            )

    rows = [row0, row1]
    sems = [si0, si1]
    r0 = wid * _ROWS_PER_W
    pltpu.make_async_copy(x_hbm.at[r0], row0, si0).start()
    pltpu.make_async_copy(x_hbm.at[r0 + 1], row1, si1).start()
    for j in range(_ROWS_PER_W):
        cur = rows[j % 2]
        pltpu.make_async_copy(x_hbm.at[r0 + j], cur, sems[j % 2]).wait()
        thresh, inv = compute_thresh_inv(cur)
        if j >= 1:
            pltpu.make_async_copy(cand, o_hbm.at[r0 + j - 1], so).wait()
        scale_to_cand(cur, thresh, inv)
        pltpu.make_async_copy(cand, o_hbm.at[r0 + j], so).start()
        if j + 2 < _ROWS_PER_W:
            pltpu.make_async_copy(x_hbm.at[r0 + j + 2], cur, sems[j % 2]).start()
    pltpu.make_async_copy(cand, o_hbm.at[r0 + _ROWS_PER_W - 1], so).wait()


@jax.jit
def kernel(x):
    mesh = plsc.VectorSubcoreMesh(
        core_axis_name="c", subcore_axis_name="s", num_cores=_NC, num_subcores=_NS
    )
    f = pl.kernel(
        _sc_kwta,
        out_type=jax.ShapeDtypeStruct((_N_ROWS, _N), jnp.float32),
        mesh=mesh,
        scratch_types=[
            pltpu.VMEM((_N,), jnp.float32),
            pltpu.VMEM((_N,), jnp.float32),
            pltpu.VMEM((_N,), jnp.float32),
            pltpu.VMEM((8192,), jnp.int32),
            pltpu.VMEM((512,), jnp.int32),
            pltpu.SemaphoreType.DMA,
            pltpu.SemaphoreType.DMA,
            pltpu.SemaphoreType.DMA,
        ],
        compiler_params=pltpu.CompilerParams(needs_layout_passes=False),
    )
    return f(x)


# speedup vs baseline: 1.1351x; 1.1074x over previous
import jax
import jax.numpy as jnp
from jax import lax
from jax.experimental import pallas as pl
from jax.experimental.pallas import tpu as pltpu
from jax.experimental.pallas import tpu_sc as plsc

_N = 32768
_N_ROWS = 128
_NC = 2
_NS = 16
_ROWS_PER_W = _N_ROWS // (_NC * _NS)
_INT_MIN = -(2**31)
_M = int(round(_N * 0.2)) - 1


def _ukey(v):
    b = plsc.bitcast(v, jnp.int32)
    m = (b >> 31) | jnp.int32(_INT_MIN)
    return plsc.bitcast(b ^ m, jnp.uint32)


def _unkey(u):
    bits = jnp.where(u >= jnp.uint32(2**31), u ^ jnp.uint32(2**31), ~u)
    return plsc.bitcast(bits, jnp.float32)


def _sc_kwta(x_hbm, o_hbm, row0, row1, cand, hist, folded, si0, si1, so):
    iota = lax.iota(jnp.int32, 16)
    ones = jnp.full((16,), 1, jnp.int32)
    zeros = jnp.zeros((16,), jnp.int32)
    wid = lax.axis_index("s") * _NC + lax.axis_index("c")

    @plsc.parallel_loop(0, 513, unroll=4)
    def _(i):
        hist[pl.ds(i * 16, 16)] = zeros

    lane15 = jnp.full((16,), 15, jnp.int32)
    lanebase = iota * 513

    def fold_and_scan(n_src, m_rem, nb):
        @plsc.parallel_loop(0, nb // 16)
        def _(c):
            acc = zeros
            for l in range(16):
                sl = pl.ds(l * 513 + c * 16, 16)
                acc = acc + hist[sl]
                hist[sl] = zeros
            folded[pl.ds(c * 16, 16)] = acc

        nv = jnp.full((16,), n_src, jnp.int32)

        def scan(c, carry):
            pc, bst, pb, hb = carry
            h = folded[pl.ds(c * 16, 16)]
            cs = plsc.cumsum(h)
            pex = pc + cs - h
            cond = (nv - pex) >= m_rem
            cpc = plsc.all_reduce_population_count(cond)
            found = cpc > 0
            ll = jnp.maximum(cpc - 1, 0)
            bst = jnp.where(found, c * 16 + ll, bst)
            pb = jnp.where(found, jnp.take(pex, ll), pb)
            hb = jnp.where(found, jnp.take(h, ll), hb)
            return pc + jnp.take(cs, lane15), bst, pb, hb

        zv = (zeros, zeros, zeros, zeros)
        _, bst, pb, hb = lax.fori_loop(0, nb // 16, scan, zv)
        return bst, m_rem - (nv - pb - hb)

    def cand_round(n_src, m_rem, shift, maskval, nb, do_compact):
        nchunks = (n_src + 15) >> 4

        @plsc.parallel_loop(0, nchunks, unroll=4)
        def _(i):
            u = plsc.bitcast(cand[pl.ds(i * 16, 16)], jnp.uint32)
            bkt = jnp.right_shift(u, jnp.uint32(shift)).astype(jnp.int32) & maskval
            lanemask = iota < (n_src - i * 16)
            plsc.addupdate_scatter(hist, [lanebase + bkt], ones, mask=lanemask)

        bst, m_rem = fold_and_scan(n_src, m_rem, nb)
        if not do_compact:
            return bst, m_rem, n_src, None

        def compactc(i, carry):
            base, acc = carry
            v = cand[pl.ds(i * 16, 16)]
            u = plsc.bitcast(v, jnp.uint32)
            bkt = jnp.right_shift(u, jnp.uint32(shift)).astype(jnp.int32) & maskval
            lanemask = iota < (n_src - i * 16)
            msk = (bkt == bst) & lanemask
            mi = msk.astype(jnp.int32)
            pos = base + plsc.cumsum(mi) - mi
            plsc.store_scatter(cand, [pos], v, mask=msk)
            acc = acc + jnp.where((bkt > bst) & lanemask, _unkey(u), 0.0)
            return base + plsc.all_reduce_population_count(msk), acc

        zf16 = jnp.zeros((16,), jnp.float32)
        base, acc = plsc.parallel_loop(0, nchunks, unroll=4, carry=(zeros, zf16))(
            compactc
        )
        return bst, m_rem, jnp.max(base), acc

    def compute_thresh_inv(row):
        @plsc.parallel_loop(0, 2048, unroll=8)
        def _(i):
            u = _ukey(row[pl.ds(i * 16, 16)])
            bkt = jnp.right_shift(u, jnp.uint32(23)).astype(jnp.int32)
            plsc.addupdate_scatter(hist, [lanebase + bkt], ones)

        b1, m_rem = fold_and_scan(_N, jnp.full((16,), _M, jnp.int32), 512)

        zf16 = jnp.zeros((16,), jnp.float32)

        def compact1(i, carry):
            base, acc = carry
            v = row[pl.ds(i * 16, 16)]
            u = _ukey(v)
            bkt = jnp.right_shift(u, jnp.uint32(23)).astype(jnp.int32)
            msk = bkt == b1
            mi = msk.astype(jnp.int32)
            pos = base + plsc.cumsum(mi) - mi
            plsc.store_scatter(cand, [pos], plsc.bitcast(u, jnp.float32), mask=msk)
            acc = acc + jnp.where(bkt > b1, v, 0.0)
            return base + plsc.all_reduce_population_count(msk), acc

        base, acc1 = plsc.parallel_loop(0, 2048, unroll=8, carry=(zeros, zf16))(
            compact1
        )
        n1 = jnp.max(base)

        b2, m_rem, n2, acc2 = cand_round(n1, m_rem, 14, 511, 512, True)

        def small_path(_):
            v = plsc.bitcast(cand[pl.ds(0, 16)], jnp.uint32)
            sk, _, _ = plsc.sort_key_val(v, v, mask=iota < n2, descending=True)
            ki = jnp.take(plsc.bitcast(sk, jnp.int32), jnp.maximum(m_rem - 1, 0))
            acc = jnp.where(iota < m_rem - 1, _unkey(sk), 0.0)
            return ki, acc, jnp.full((16,), _M - 1, jnp.int32)

        def big_path(_):
            b3, m_rem3, n3, acc3 = cand_round(n2, m_rem, 5, 511, 512, True)
            b4, m_rem4, _, _ = cand_round(n3, m_rem3, 0, 31, 32, False)
            ki = (b1 << 23) | (b2 << 14) | (b3 << 5) | b4
            kuv = plsc.bitcast(ki, jnp.uint32)

            def tail(i, acc):
                u = plsc.bitcast(cand[pl.ds(i * 16, 16)], jnp.uint32)
                sel = (u > kuv) & (iota < (n3 - i * 16))
                return acc + jnp.where(sel, _unkey(u), 0.0)

            acc4 = plsc.parallel_loop(0, (n3 + 15) >> 4, unroll=4, carry=zf16)(tail)
            return ki, acc3 + acc4, _M - m_rem4

        ki, acc3, cnt = lax.cond(n2 <= 16, small_path, big_path, 0)
        kv = plsc.bitcast(ki, jnp.uint32)
        tval = _unkey(kv)
        thresh = tval + jnp.float32(1e-8)

        s = jnp.sum(acc1 + acc2 + acc3)
        sv = jnp.full((16,), s) - cnt.astype(jnp.float32) * thresh
        inv = 1.0 / (sv + jnp.float32(1e-8))
        return thresh, inv

    def scale_to_cand(row, thresh, inv):
        @plsc.parallel_loop(0, 2048, unroll=8)
        def _(i):
            cand[pl.ds(i * 16, 16)] = (
                jnp.maximum(row[pl.ds(i * 16, 16)] - thresh, 0.0) * inv
            )

    rows = [row0, row1]
    sems = [si0, si1]
    r0 = wid * _ROWS_PER_W
    pltpu.make_async_copy(x_hbm.at[r0], row0, si0).start()
    pltpu.make_async_copy(x_hbm.at[r0 + 1], row1, si1).start()
    for j in range(_ROWS_PER_W):
        cur = rows[j % 2]
        pltpu.make_async_copy(x_hbm.at[r0 + j], cur, sems[j % 2]).wait()
        thresh, inv = compute_thresh_inv(cur)
        if j >= 1:
            pltpu.make_async_copy(cand, o_hbm.at[r0 + j - 1], so).wait()
        scale_to_cand(cur, thresh, inv)
        pltpu.make_async_copy(cand, o_hbm.at[r0 + j], so).start()
        if j + 2 < _ROWS_PER_W:
            pltpu.make_async_copy(x_hbm.at[r0 + j + 2], cur, sems[j % 2]).start()
    pltpu.make_async_copy(cand, o_hbm.at[r0 + _ROWS_PER_W - 1], so).wait()


@jax.jit
def kernel(x):
    mesh = plsc.VectorSubcoreMesh(
        core_axis_name="c", subcore_axis_name="s", num_cores=_NC, num_subcores=_NS
    )
    f = pl.kernel(
        _sc_kwta,
        out_type=jax.ShapeDtypeStruct((_N_ROWS, _N), jnp.float32),
        mesh=mesh,
        scratch_types=[
            pltpu.VMEM((_N,), jnp.float32),
            pltpu.VMEM((_N,), jnp.float32),
            pltpu.VMEM((_N,), jnp.float32),
            pltpu.VMEM((8208,), jnp.int32),
            pltpu.VMEM((512,), jnp.int32),
            pltpu.SemaphoreType.DMA,
            pltpu.SemaphoreType.DMA,
            pltpu.SemaphoreType.DMA,
        ],
        compiler_params=pltpu.CompilerParams(needs_layout_passes=False),
    )
    return f(x)
